# baseline (speedup 1.0000x reference)
import jax
import jax.numpy as jnp
from jax import lax
from jax.experimental import pallas as pl
from jax.experimental.pallas import tpu as pltpu
from jax.experimental.pallas import tpu_sc as plsc

B = 4096 * 26
D = 64
NC, NS = 2, 16
NW = NC * NS
BPW = B // NW
CH = 832
NCHUNK = BPW // CH


def _emb_body(idx_hbm, tab_hbm, out_hbm, idx_v, sem_i, sem_r):
    wid = lax.axis_index("s") * NC + lax.axis_index("c")
    base = wid * BPW
    pltpu.async_copy(idx_hbm.at[pl.ds(base, BPW)], idx_v, sem_i).wait()

    @pl.loop(0, BPW, step=16)
    def _row(j):
        vec = idx_v[pl.ds(j, 16)]
        for k in range(16):
            pltpu.async_copy(
                tab_hbm.at[pl.ds(vec[k], 1)],
                out_hbm.at[pl.ds(base + j + k, 1)], sem_r)

    pltpu.make_async_copy(
        tab_hbm.at[pl.ds(0, BPW)], out_hbm.at[pl.ds(base, BPW)], sem_r
    ).wait()


def kernel(input, table):
    idx = input.reshape(-1).astype(jnp.int32)
    mesh = plsc.VectorSubcoreMesh(core_axis_name="c", subcore_axis_name="s")
    k = pl.kernel(
        _emb_body,
        out_type=jax.ShapeDtypeStruct((B, D), jnp.float32),
        mesh=mesh,
        scratch_types=[
            pltpu.VMEM((BPW,), jnp.int32),
            pltpu.SemaphoreType.DMA,
            pltpu.SemaphoreType.DMA,
        ],
    )
    out = k(idx, table)
    return out.reshape(input.shape + (D,))


# speedup vs baseline: 3.1002x; 3.1002x over previous
import jax
import jax.numpy as jnp
from jax import lax
from jax.experimental import pallas as pl
from jax.experimental.pallas import tpu as pltpu
from jax.experimental.pallas import tpu_sc as plsc

V = 1_000_000
D = 64
DP = 128
B = 4096 * 26
NC, NS = 2, 16
NW = NC * NS
BPW = B // NW
C = 416
NGCH = BPW // C


def _gather_body(idx_hbm, pack_hbm, out_hbm, idx_v, rows0, rows1, semi,
                 sem0, sem1):
    wid = lax.axis_index("s") * NC + lax.axis_index("c")
    base = wid * BPW
    pltpu.async_copy(idx_hbm.at[pl.ds(base, BPW)], idx_v, semi).wait()
    bufs, sems = (rows0, rows1), (sem0, sem1)
    ins = [None, None]
    ins[0] = pltpu.async_copy(pack_hbm.at[idx_v.at[pl.ds(0, C)]], rows0, sem0)
    for c in range(NGCH):
        cur = c % 2
        if c + 1 < NGCH:
            nxt = (c + 1) % 2
            ins[nxt] = pltpu.async_copy(
                pack_hbm.at[idx_v.at[pl.ds((c + 1) * C, C)]], bufs[nxt],
                sems[nxt])
        ins[cur].wait()
        pltpu.sync_copy(bufs[cur], out_hbm.at[pl.ds(base + c * C, C)])


def kernel(input, table):
    idx = input.reshape(-1).astype(jnp.int32)
    tabp = jnp.pad(table, ((0, 0), (0, DP - D)))
    mesh = plsc.VectorSubcoreMesh(core_axis_name="c", subcore_axis_name="s")
    wide = pl.kernel(
        _gather_body,
        out_type=jax.ShapeDtypeStruct((B, DP), jnp.float32),
        mesh=mesh,
        compiler_params=pltpu.CompilerParams(needs_layout_passes=False),
        scratch_types=[
            pltpu.VMEM((BPW,), jnp.int32),
            pltpu.VMEM((C, DP), jnp.float32),
            pltpu.VMEM((C, DP), jnp.float32),
            pltpu.SemaphoreType.DMA,
            pltpu.SemaphoreType.DMA,
            pltpu.SemaphoreType.DMA,
        ],
    )(idx, tabp)
    return wide[:, :D].reshape(input.shape + (D,))
